# Initial kernel scaffold; baseline (speedup 1.0000x reference)
import functools

import jax
import jax.numpy as jnp
from jax import lax
from jax.experimental import pallas as pl
from jax.experimental.pallas import tpu as pltpu
from jax.experimental.pallas import tpu_sc as plsc

N = 10000
E = 320000
EPAD = 327680
RPT = 80
ACC_ROWS = 10240
ZPT = ACC_ROWS // 16
WPT = N // 16


def _make_deg():
    mesh = plsc.VectorSubcoreMesh(core_axis_name="c", subcore_axis_name="s")

    @functools.partial(
        pl.kernel, mesh=mesh,
        out_type=jax.ShapeDtypeStruct((2 * N, 16), jnp.float32),
        scratch_types=[
            pltpu.VMEM((RPT, 128), jnp.int32),
            pltpu.VMEM((128, 16), jnp.float32),
        ],
    )
    def deg(row_hbm, ones_hbm, zeros_hbm, out_hbm, idx_v, ones_v):
        c = lax.axis_index("c")
        s = lax.axis_index("s")
        wid = c * 16 + s

        def run(acc):
            pltpu.sync_copy(zeros_hbm.at[pl.ds(s * ZPT, ZPT)],
                            acc.at[pl.ds(s * ZPT, ZPT)])
            pltpu.sync_copy(ones_hbm, ones_v)
            pltpu.sync_copy(row_hbm.at[pl.ds(wid * RPT, RPT)], idx_v)
            plsc.subcore_barrier()

            def body(j, carry):
                pltpu.sync_copy(ones_v, acc.at[idx_v.at[j]], add=True)
                return carry

            lax.fori_loop(0, RPT, body, 0)
            plsc.subcore_barrier()
            pltpu.sync_copy(acc.at[pl.ds(s * WPT, WPT)],
                            out_hbm.at[pl.ds(c * N + s * WPT, WPT)])

        pl.run_scoped(run, pltpu.VMEM_SHARED((ACC_ROWS, 16), jnp.float32))

    return deg


def _make_prop(D):
    mesh = plsc.VectorSubcoreMesh(core_axis_name="c", subcore_axis_name="s")

    @functools.partial(
        pl.kernel, mesh=mesh,
        out_type=jax.ShapeDtypeStruct((2 * N, D), jnp.float32),
        scratch_types=[
            pltpu.VMEM((RPT, 128), jnp.int32),
            pltpu.VMEM((RPT, 128), jnp.int32),
            pltpu.VMEM((128, D), jnp.float32),
            pltpu.SemaphoreType.DMA,
        ],
    )
    def prop(row_hbm, col_hbm, z_hbm, zeros_hbm, out_hbm,
             row_v, col_v, buf, sem):
        c = lax.axis_index("c")
        s = lax.axis_index("s")
        wid = c * 16 + s

        def run(acc):
            pltpu.sync_copy(zeros_hbm.at[pl.ds(s * ZPT, ZPT)],
                            acc.at[pl.ds(s * ZPT, ZPT)])
            pltpu.sync_copy(row_hbm.at[pl.ds(wid * RPT, RPT)], row_v)
            pltpu.sync_copy(col_hbm.at[pl.ds(wid * RPT, RPT)], col_v)
            plsc.subcore_barrier()

            def body(j, carry):
                pltpu.async_copy(z_hbm.at[row_v.at[j]], buf, sem).wait()
                pltpu.sync_copy(buf, acc.at[col_v.at[j]], add=True)
                return carry

            lax.fori_loop(0, RPT, body, 0)
            plsc.subcore_barrier()
            pltpu.sync_copy(acc.at[pl.ds(s * WPT, WPT)],
                            out_hbm.at[pl.ds(c * N + s * WPT, WPT)])

        pl.run_scoped(run, pltpu.VMEM_SHARED((ACC_ROWS, D), jnp.float32))

    return prop


_deg = _make_deg()
_prop128 = _make_prop(128)
_prop64 = _make_prop(64)

_B = 2000


def _rs(dA, dB):
    return lax.rsqrt(dA[:, :1] + dB[:, :1] + 1.0)


def _tc1_body(x_ref, w_ref, b_ref, dA_ref, dB_ref, z_ref):
    s = _rs(dA_ref[...], dB_ref[...])
    y = lax.dot_general(x_ref[...], w_ref[...], (((1,), (1,)), ((), ())),
                        preferred_element_type=jnp.float32)
    z_ref[...] = s * (y + b_ref[...])


def _tc2_body(aA_ref, aB_ref, z1_ref, dA_ref, dB_ref, w_ref, b_ref, z2_ref):
    s = _rs(dA_ref[...], dB_ref[...])
    h = jnp.maximum(s * (aA_ref[...] + aB_ref[...] + z1_ref[...]), 0.0)
    y = lax.dot_general(h, w_ref[...], (((1,), (1,)), ((), ())),
                        preferred_element_type=jnp.float32)
    z2_ref[...] = s * (y + b_ref[...])


def _tc3_body(aA_ref, aB_ref, z2_ref, dA_ref, dB_ref, out_ref):
    s = _rs(dA_ref[...], dB_ref[...])
    o = s * (aA_ref[...] + aB_ref[...] + z2_ref[...])
    m = jnp.max(o, axis=1, keepdims=True)
    e = o - m
    out_ref[...] = e - jnp.log(jnp.sum(jnp.exp(e), axis=1, keepdims=True))


def _row_spec(d):
    return pl.BlockSpec((_B, d), lambda i: (i, 0))


def _full_spec(r, d):
    return pl.BlockSpec((r, d), lambda i: (0, 0))


def _tc1(x, W1, b1, dA, dB):
    return pl.pallas_call(
        _tc1_body, grid=(N // _B,),
        in_specs=[_row_spec(128), _full_spec(128, 128), _full_spec(1, 128),
                  _row_spec(16), _row_spec(16)],
        out_specs=_row_spec(128),
        out_shape=jax.ShapeDtypeStruct((N, 128), jnp.float32),
    )(x, W1, b1, dA, dB)


def _tc2(aA, aB, z1, dA, dB, W2, b2):
    return pl.pallas_call(
        _tc2_body, grid=(N // _B,),
        in_specs=[_row_spec(128), _row_spec(128), _row_spec(128),
                  _row_spec(16), _row_spec(16),
                  _full_spec(64, 128), _full_spec(1, 64)],
        out_specs=_row_spec(64),
        out_shape=jax.ShapeDtypeStruct((N, 64), jnp.float32),
    )(aA, aB, z1, dA, dB, W2, b2)


def _tc3(aA, aB, z2, dA, dB):
    return pl.pallas_call(
        _tc3_body, grid=(N // _B,),
        in_specs=[_row_spec(64), _row_spec(64), _row_spec(64),
                  _row_spec(16), _row_spec(16)],
        out_specs=_row_spec(64),
        out_shape=jax.ShapeDtypeStruct((N, 64), jnp.float32),
    )(aA, aB, z2, dA, dB)


def kernel(x, edge_index, W1, b1, W2, b2):
    ei = edge_index.astype(jnp.int32)
    row, col = ei[0], ei[1]
    pad = EPAD - E
    row_deg = jnp.concatenate([row, jnp.full((pad,), N, jnp.int32)])
    row_g = jnp.concatenate([row, jnp.zeros((pad,), jnp.int32)])
    col_s = jnp.concatenate([col, jnp.full((pad,), N, jnp.int32)])
    row_deg = row_deg.reshape(EPAD // 128, 128)
    row_g = row_g.reshape(EPAD // 128, 128)
    col_s = col_s.reshape(EPAD // 128, 128)

    ones16 = jnp.ones((128, 16), jnp.float32)
    zeros16 = jnp.zeros((ACC_ROWS, 16), jnp.float32)
    zeros128 = jnp.zeros((ACC_ROWS, 128), jnp.float32)
    zeros64 = jnp.zeros((ACC_ROWS, 64), jnp.float32)

    deg2 = _deg(row_deg, ones16, zeros16)
    dA, dB = deg2[:N], deg2[N:]

    z1 = _tc1(x, W1, b1.reshape(1, 128), dA, dB)
    agg1 = _prop128(row_g, col_s, z1, zeros128)
    z2 = _tc2(agg1[:N], agg1[N:], z1, dA, dB, W2, b2.reshape(1, 64))
    agg2 = _prop64(row_g, col_s, z2, zeros64)
    return _tc3(agg2[:N], agg2[N:], z2, dA, dB)


# speedup vs baseline: 8.2623x; 8.2623x over previous
import functools

import jax
import jax.numpy as jnp
from jax import lax
from jax.experimental import pallas as pl
from jax.experimental.pallas import tpu as pltpu
from jax.experimental.pallas import tpu_sc as plsc

N = 10000
E = 320000
EPAD = 327680
RPT = 80
ACC_ROWS = 10240
ZPT = ACC_ROWS // 16


def _make_deg():
    mesh = plsc.VectorSubcoreMesh(core_axis_name="c", subcore_axis_name="s")

    @functools.partial(
        pl.kernel, mesh=mesh,
        out_type=jax.ShapeDtypeStruct((2 * ACC_ROWS, 128), jnp.float32),
        scratch_types=[
            pltpu.VMEM((RPT, 128), jnp.int32),
            pltpu.VMEM((128, 128), jnp.float32),
            pltpu.VMEM_SHARED((ACC_ROWS, 128), jnp.float32),
        ],
    )
    def deg(row_hbm, ones_hbm, zeros_hbm, out_hbm, idx_v, ones_v, acc):
        c = lax.axis_index("c")
        s = lax.axis_index("s")
        wid = c * 16 + s

        pltpu.sync_copy(zeros_hbm.at[pl.ds(s * ZPT, ZPT)],
                        acc.at[pl.ds(s * ZPT, ZPT)])
        pltpu.sync_copy(ones_hbm, ones_v)
        pltpu.sync_copy(row_hbm.at[pl.ds(wid * RPT, RPT)], idx_v)
        plsc.subcore_barrier()

        def body(j, carry):
            pltpu.sync_copy(ones_v, acc.at[idx_v.at[j]], add=True)
            return carry

        lax.fori_loop(0, RPT, body, 0)
        plsc.subcore_barrier()
        pltpu.sync_copy(acc.at[pl.ds(s * ZPT, ZPT)],
                        out_hbm.at[pl.ds(c * ACC_ROWS + s * ZPT, ZPT)])

    return deg


def _make_prop(D):
    mesh = plsc.VectorSubcoreMesh(core_axis_name="c", subcore_axis_name="s")

    @functools.partial(
        pl.kernel, mesh=mesh,
        out_type=jax.ShapeDtypeStruct((2 * ACC_ROWS, D), jnp.float32),
        scratch_types=[
            pltpu.VMEM((RPT, 128), jnp.int32),
            pltpu.VMEM((RPT, 128), jnp.int32),
            pltpu.VMEM((128, D), jnp.float32),
            pltpu.VMEM_SHARED((ACC_ROWS, D), jnp.float32),
            pltpu.SemaphoreType.DMA,
        ],
    )
    def prop(row_hbm, col_hbm, z_hbm, zeros_hbm, out_hbm,
             row_v, col_v, buf, acc, sem):
        c = lax.axis_index("c")
        s = lax.axis_index("s")
        wid = c * 16 + s

        if True:
            pltpu.sync_copy(zeros_hbm.at[pl.ds(s * ZPT, ZPT)],
                            acc.at[pl.ds(s * ZPT, ZPT)])
            pltpu.sync_copy(row_hbm.at[pl.ds(wid * RPT, RPT)], row_v)
            pltpu.sync_copy(col_hbm.at[pl.ds(wid * RPT, RPT)], col_v)
            plsc.subcore_barrier()

            def body(j, carry):
                pltpu.async_copy(z_hbm.at[row_v.at[j]], buf, sem).wait()
                pltpu.sync_copy(buf, acc.at[col_v.at[j]], add=True)
                return carry

            lax.fori_loop(0, RPT, body, 0)
            plsc.subcore_barrier()
            pltpu.sync_copy(acc.at[pl.ds(s * ZPT, ZPT)],
                            out_hbm.at[pl.ds(c * ACC_ROWS + s * ZPT, ZPT)])


    return prop


_deg = _make_deg()
_prop128 = _make_prop(128)

_B = 2000


def _rs(dA, dB):
    return lax.rsqrt(dA[:, :1] + dB[:, :1] + 1.0)


def _tc1_body(x_ref, w_ref, b_ref, dA_ref, dB_ref, z_ref):
    s = _rs(dA_ref[...], dB_ref[...])
    y = lax.dot_general(x_ref[...], w_ref[...], (((1,), (1,)), ((), ())),
                        preferred_element_type=jnp.float32)
    z_ref[...] = s * (y + b_ref[...])


def _tc2_body(aA_ref, aB_ref, z1_ref, dA_ref, dB_ref, w_ref, b_ref, z2_ref):
    s = _rs(dA_ref[...], dB_ref[...])
    h = jnp.maximum(s * (aA_ref[...] + aB_ref[...] + z1_ref[...]), 0.0)
    y = lax.dot_general(h, w_ref[...], (((1,), (1,)), ((), ())),
                        preferred_element_type=jnp.float32)
    z2_ref[...] = s * (y + b_ref[...])


def _tc3_body(aA_ref, aB_ref, z2_ref, dA_ref, dB_ref, out_ref):
    s = _rs(dA_ref[...], dB_ref[...])
    o = s * (aA_ref[:, :64] + aB_ref[:, :64] + z2_ref[:, :64])
    m = jnp.max(o, axis=1, keepdims=True)
    e = o - m
    out_ref[...] = e - jnp.log(jnp.sum(jnp.exp(e), axis=1, keepdims=True))


def _row_spec(d):
    return pl.BlockSpec((_B, d), lambda i: (i, 0))


def _full_spec(r, d):
    return pl.BlockSpec((r, d), lambda i: (0, 0))


def _tc1(x, W1, b1, dA, dB):
    return pl.pallas_call(
        _tc1_body, grid=(N // _B,),
        in_specs=[_row_spec(128), _full_spec(128, 128), _full_spec(1, 128),
                  _row_spec(128), _row_spec(128)],
        out_specs=_row_spec(128),
        out_shape=jax.ShapeDtypeStruct((N, 128), jnp.float32),
    )(x, W1, b1, dA, dB)


def _tc2(aA, aB, z1, dA, dB, W2p, b2p):
    return pl.pallas_call(
        _tc2_body, grid=(N // _B,),
        in_specs=[_row_spec(128), _row_spec(128), _row_spec(128),
                  _row_spec(128), _row_spec(128),
                  _full_spec(128, 128), _full_spec(1, 128)],
        out_specs=_row_spec(128),
        out_shape=jax.ShapeDtypeStruct((N, 128), jnp.float32),
    )(aA, aB, z1, dA, dB, W2p, b2p)


def _tc3(aA, aB, z2, dA, dB):
    return pl.pallas_call(
        _tc3_body, grid=(N // _B,),
        in_specs=[_row_spec(128), _row_spec(128), _row_spec(128),
                  _row_spec(128), _row_spec(128)],
        out_specs=_row_spec(64),
        out_shape=jax.ShapeDtypeStruct((N, 64), jnp.float32),
    )(aA, aB, z2, dA, dB)


def kernel(x, edge_index, W1, b1, W2, b2):
    ei = edge_index.astype(jnp.int32)
    row, col = ei[0], ei[1]
    pad = EPAD - E
    row_deg = jnp.concatenate([row, jnp.full((pad,), N, jnp.int32)])
    row_g = jnp.concatenate([row, jnp.zeros((pad,), jnp.int32)])
    col_s = jnp.concatenate([col, jnp.full((pad,), N, jnp.int32)])
    row_deg = row_deg.reshape(EPAD // 128, 128)
    row_g = row_g.reshape(EPAD // 128, 128)
    col_s = col_s.reshape(EPAD // 128, 128)

    ones128 = jnp.ones((128, 128), jnp.float32)
    zeros128 = jnp.zeros((ACC_ROWS, 128), jnp.float32)
    W2p = jnp.concatenate([W2, jnp.zeros((64, 128), jnp.float32)], axis=0)
    b2p = jnp.concatenate([b2, jnp.zeros((64,), jnp.float32)]).reshape(1, 128)

    deg2 = _deg(row_deg, ones128, zeros128)
    dA, dB = deg2[:N], deg2[ACC_ROWS:ACC_ROWS + N]

    z1 = _tc1(x, W1, b1.reshape(1, 128), dA, dB)
    agg1 = _prop128(row_g, col_s, z1, zeros128)
    z2 = _tc2(agg1[:N], agg1[ACC_ROWS:ACC_ROWS + N], z1, dA, dB, W2p, b2p)
    agg2 = _prop128(row_g, col_s, z2, zeros128)
    return _tc3(agg2[:N], agg2[ACC_ROWS:ACC_ROWS + N], z2, dA, dB)
